# baseline (speedup 1.0000x reference)
import jax
import jax.numpy as jnp
from jax.experimental import pallas as pl
from jax.experimental.pallas import tpu as pltpu


def _half_pool(x_ref, o_ref, o_base, tp):
    _, _, d, h, w = x_ref.shape
    for k in range(tp):
        m = None
        for t in range(2):
            for dd in range(2):
                for hh in range(2):
                    v = x_ref[
                        pl.ds(0, 1), pl.ds(2 * k + t, 1),
                        pl.ds(dd, d // 2, 2), pl.ds(hh, h // 2, 2), :,
                    ].reshape(d // 2, h // 2, w)
                    m = v if m is None else jnp.maximum(m, v)
        p = jnp.maximum(m, pltpu.roll(m, w - 1, axis=2))
        idx = 2 * jax.lax.broadcasted_iota(
            jnp.int32, (d // 2, h // 2, w // 2), 2
        )
        o_ref[0, o_base + k] = jnp.take_along_axis(p, idx, axis=2)


def _pool_body(xa_ref, xb_ref, o_ref):
    tp = xa_ref.shape[1] // 2
    _half_pool(xa_ref, o_ref, 0, tp)
    _half_pool(xb_ref, o_ref, tp, tp)


def kernel(x):
    b, c, t, d, h, w = x.shape
    xr = x.reshape(b * c, t, d, h, w)
    th = t // 2
    out = pl.pallas_call(
        _pool_body,
        grid=(b * c,),
        in_specs=[
            pl.BlockSpec((1, th, d, h, w), lambda i: (i, 0, 0, 0, 0)),
            pl.BlockSpec((1, th, d, h, w), lambda i: (i, 1, 0, 0, 0)),
        ],
        out_specs=pl.BlockSpec(
            (1, t // 2, d // 2, h // 2, w // 2), lambda i: (i, 0, 0, 0, 0)
        ),
        out_shape=jax.ShapeDtypeStruct(
            (b * c, t // 2, d // 2, h // 2, w // 2), x.dtype
        ),
        compiler_params=pltpu.CompilerParams(
            dimension_semantics=("parallel",),
        ),
    )(xr, xr)
    return out.reshape(b, c, t // 2, d // 2, h // 2, w // 2)
